# Initial kernel scaffold; baseline (speedup 1.0000x reference)
import jax
import jax.numpy as jnp
from jax.experimental import pallas as pl


def kernel(x, table):
    raise NotImplementedError("write your pallas kernel here")


# speedup vs baseline: 2.5799x; 2.5799x over previous
import functools

import jax
import jax.numpy as jnp
from jax import lax
from jax.experimental import pallas as pl
from jax.experimental.pallas import tpu as pltpu
from jax.experimental.pallas import tpu_sc as plsc

EMB = 32
L_OUT = 200
NW = 32
B_PAD = 256
B_PER_W = B_PAD // NW


def _make_gather(vocab: int):
    mesh = plsc.VectorSubcoreMesh(core_axis_name="c", subcore_axis_name="s")

    @functools.partial(
        pl.kernel,
        mesh=mesh,
        out_type=jax.ShapeDtypeStruct((B_PAD, EMB), jnp.float32),
        scratch_types=[
            pltpu.VMEM((B_PER_W,), jnp.int32),
            pltpu.VMEM((B_PER_W, EMB), jnp.float32),
            pltpu.SemaphoreType.DMA,
        ],
        compiler_params=pltpu.CompilerParams(use_tc_tiling_on_sc=False),
    )
    def gather_kernel(table_hbm, idx_hbm, out_hbm, idx_v, rows_v, sem):
        wid = lax.axis_index("s") * 2 + lax.axis_index("c")
        base = wid * B_PER_W
        pltpu.sync_copy(idx_hbm.at[pl.ds(base, B_PER_W)], idx_v)
        pltpu.async_copy(table_hbm.at[idx_v], rows_v, sem).wait()
        pltpu.sync_copy(rows_v, out_hbm.at[pl.ds(base, B_PER_W)])

    return gather_kernel


def kernel(x, table):
    idx = jnp.pad(x[0], (0, B_PAD - L_OUT))
    out = _make_gather(table.shape[0])(table, idx)
    return out[:L_OUT]


# speedup vs baseline: 47.6803x; 18.4814x over previous
import functools

import jax
import jax.numpy as jnp
from jax import lax
from jax.experimental import pallas as pl
from jax.experimental.pallas import tpu as pltpu
from jax.experimental.pallas import tpu_sc as plsc

EMB = 32
L_OUT = 200
NW = 32
B_PAD = 256
B_PER_W = B_PAD // NW
IDX_PAD = B_PAD + 8
LANES = 128


def _make_gather():
    mesh = plsc.VectorSubcoreMesh(core_axis_name="c", subcore_axis_name="s")

    @functools.partial(
        pl.kernel,
        mesh=mesh,
        out_type=jax.ShapeDtypeStruct((B_PAD, EMB), jnp.float32),
        scratch_types=[
            pltpu.VMEM((16,), jnp.int32),
            pltpu.VMEM((B_PER_W, EMB, LANES), jnp.float32),
            pltpu.VMEM((B_PER_W, EMB), jnp.float32),
            pltpu.SemaphoreType.DMA,
        ],
        compiler_params=pltpu.CompilerParams(needs_layout_passes=False),
    )
    def gather_kernel(tableT_hbm, idx_hbm, out_hbm, idx_v, blk_v, rows_v, sem):
        wid = lax.axis_index("s") * 2 + lax.axis_index("c")
        base = wid * B_PER_W
        pltpu.sync_copy(idx_hbm.at[pl.ds(base, 16)], idx_v)
        vv = idx_v[...]
        copies = []
        for j in range(B_PER_W):
            ib = pl.multiple_of((vv[j] // LANES) * LANES, LANES)
            copies.append(
                pltpu.async_copy(
                    tableT_hbm.at[:, pl.ds(ib, LANES)], blk_v.at[j], sem
                )
            )
        for c in copies:
            c.wait()
        iota = lax.iota(jnp.int32, 16)
        for j in range(B_PER_W):
            lane = jnp.full((16,), vv[j] % LANES, jnp.int32)
            for h in range(EMB // 16):
                vals = plsc.load_gather(blk_v.at[j], [iota + 16 * h, lane])
                rows_v[j, pl.ds(16 * h, 16)] = vals
        pltpu.sync_copy(rows_v, out_hbm.at[pl.ds(base, B_PER_W)])

    return gather_kernel


def kernel(x, table):
    idx = jnp.pad(x[0], (0, IDX_PAD - L_OUT))
    out = _make_gather()(table.T, idx)
    return out[:L_OUT]


# speedup vs baseline: 53.3875x; 1.1197x over previous
import functools

import jax
import jax.numpy as jnp
from jax import lax
from jax.experimental import pallas as pl
from jax.experimental.pallas import tpu as pltpu
from jax.experimental.pallas import tpu_sc as plsc

EMB = 32
L_OUT = 200
NW = 32
B_PER_W = 8
B_PAD = NW * B_PER_W
LANES = 128


def _make_gather():
    mesh = plsc.VectorSubcoreMesh(core_axis_name="c", subcore_axis_name="s")

    @functools.partial(
        pl.kernel,
        mesh=mesh,
        out_type=jax.ShapeDtypeStruct((B_PAD, EMB), jnp.float32),
        scratch_types=[
            pltpu.VMEM((16, LANES), jnp.int32),
            pltpu.VMEM((B_PER_W, EMB, LANES), jnp.float32),
            pltpu.VMEM((B_PER_W, EMB), jnp.float32),
            pltpu.SemaphoreType.DMA,
        ],
        compiler_params=pltpu.CompilerParams(needs_layout_passes=False),
    )
    def gather_kernel(tableT_hbm, xT_hbm, out_hbm, idx_v, blk_v, rows_v, sem):
        wid = lax.axis_index("s") * 2 + lax.axis_index("c")
        base = wid * B_PER_W
        rb = jnp.minimum(base, L_OUT - 16)
        d = jnp.minimum(base - rb, 8)
        pltpu.sync_copy(xT_hbm.at[pl.ds(rb, 16), pl.ds(0, LANES)], idx_v)
        zeros16 = jnp.zeros((16,), jnp.int32)
        iota = lax.iota(jnp.int32, 16)

        def fire(j, carry):
            vj = plsc.load_gather(
                idx_v, [jnp.full((16,), d + j, jnp.int32), zeros16]
            )
            ib = pl.multiple_of((vj[0] // LANES) * LANES, LANES)
            pltpu.async_copy(
                tableT_hbm.at[:, pl.ds(ib, LANES)], blk_v.at[j], sem
            )
            return carry

        lax.fori_loop(0, B_PER_W, fire, 0)

        def drain(j, carry):
            pltpu.make_async_copy(
                tableT_hbm.at[:, pl.ds(0, LANES)], blk_v.at[j], sem
            ).wait()
            return carry

        lax.fori_loop(0, B_PER_W, drain, 0)

        def extract(j, carry):
            vj = plsc.load_gather(
                idx_v, [jnp.full((16,), d + j, jnp.int32), zeros16]
            )
            lane = vj % LANES
            for h in range(EMB // 16):
                vals = plsc.load_gather(blk_v.at[j], [iota + 16 * h, lane])
                plsc.store_scatter(
                    rows_v,
                    [jnp.full((16,), j, jnp.int32), iota + 16 * h],
                    vals,
                )
            return carry

        lax.fori_loop(0, B_PER_W, extract, 0)
        pltpu.sync_copy(rows_v, out_hbm.at[pl.ds(base, B_PER_W)])

    return gather_kernel


def kernel(x, table):
    out = _make_gather()(table.T, x.T)
    return out[:L_OUT]
